# Initial kernel scaffold; baseline (speedup 1.0000x reference)
import jax
import jax.numpy as jnp
from jax.experimental import pallas as pl


def kernel(x, Wq, bq, Wk, bk, Wv, bv, Wo, bo, edge_weight, W1, b1, W2, b2):
    raise NotImplementedError("write your pallas kernel here")


# speedup vs baseline: 4.9255x; 4.9255x over previous
import functools

import numpy as np
import jax
import jax.numpy as jnp
from jax import lax
from jax.experimental import pallas as pl
from jax.experimental.pallas import tpu as pltpu
from jax.experimental.pallas import tpu_sc as plsc

_B, _S, _D, _H, _DH = 2, 8192, 768, 12, 64
_E = 500
_EP = 512
_NW = 32
_GPW = (_B * 2 * _EP) // _NW
_RPW = (_B * _S) // _NW
_K = 32
_SCALE = _DH ** -0.5


def _halton(b, n):
    h, d = 0, 1
    seq = []
    for _ in range(n):
        x = d - h
        if x == 1:
            h = 1
            d *= b
        else:
            y = d // b
            while x <= y:
                y //= b
            h = (b + 1) * y - x
        seq.append(h / d)
    return np.array(seq, dtype=np.float64)


def _build_constants():
    n = min(500, int(0.01 * _S * _S))
    h2 = _halton(2, n)
    h3 = _halton(3, n)
    src = (h2 * _S).astype(np.int64)
    dst = (h3 * _S).astype(np.int64)
    keep = src != dst
    src = src[keep][:n]
    dst = dst[keep][:n]
    assert src.shape[0] == _E
    assert np.unique(src).size == _E

    gidx = np.zeros((_B, 2, _EP), dtype=np.int32)
    for b in range(_B):
        gidx[b, 0, :_E] = b * _S + src
        gidx[b, 0, _E:] = b * _S
        gidx[b, 1, :_E] = b * _S + dst
        gidx[b, 1, _E:] = b * _S
    gidx = gidx.reshape(-1)

    didx = np.zeros((_NW, _K), dtype=np.int32)
    sidx = np.zeros((_NW, _K), dtype=np.int32)
    for t in range(_NW):
        lo = t * _RPW
        rows = []
        for b in range(_B):
            flat = b * _S + src
            for e in np.nonzero((flat >= lo) & (flat < lo + _RPW))[0]:
                rows.append((b * _EP + int(e), int(flat[e])))
        assert 0 < len(rows) <= _K
        while len(rows) < _K:
            rows.append(rows[-1])
        didx[t] = [r[0] for r in rows]
        sidx[t] = [r[1] for r in rows]
    return gidx, didx.reshape(-1), sidx.reshape(-1)


_GIDX_NP, _DIDX_NP, _SIDX_NP = _build_constants()

def _sc_gather_body(x_hbm, idx_hbm, out_hbm, idx_v, rows_v, sem):
    wid = lax.axis_index("s") * 2 + lax.axis_index("c")
    base = wid * _GPW
    pltpu.sync_copy(idx_hbm.at[pl.ds(base, _GPW)], idx_v)
    pltpu.async_copy(x_hbm.at[idx_v], rows_v, sem).wait()
    pltpu.sync_copy(rows_v, out_hbm.at[pl.ds(base, _GPW)])


def _sc_fill_scatter_body(fill_hbm, delta_hbm, didx_hbm, sidx_hbm, out_hbm,
                          fill_v, didx_v, sidx_v, drows_v, fsem, ssem):
    wid = lax.axis_index("s") * 2 + lax.axis_index("c")
    base = wid * _RPW
    pltpu.sync_copy(fill_hbm, fill_v)
    fills = [
        pltpu.async_copy(fill_v, out_hbm.at[pl.ds(base + j * 64, 64)], fsem)
        for j in range(_RPW // 64)
    ]
    pltpu.sync_copy(didx_hbm.at[pl.ds(wid * _K, _K)], didx_v)
    pltpu.sync_copy(sidx_hbm.at[pl.ds(wid * _K, _K)], sidx_v)
    pltpu.async_copy(delta_hbm.at[didx_v], drows_v, ssem).wait()
    for c in fills:
        c.wait()
    pltpu.async_copy(drows_v, out_hbm.at[sidx_v], ssem).wait()


@functools.lru_cache(maxsize=None)
def _sc_kernels():
    mesh = plsc.VectorSubcoreMesh(core_axis_name="c", subcore_axis_name="s")
    gather = pl.kernel(
        _sc_gather_body,
        out_type=jax.ShapeDtypeStruct((_B * 2 * _EP, _D), jnp.float32),
        mesh=mesh,
        scratch_types=[
            pltpu.VMEM((_GPW,), jnp.int32),
            pltpu.VMEM((_GPW, _D), jnp.float32),
            pltpu.SemaphoreType.DMA,
        ],
    )
    fill_scatter = pl.kernel(
        _sc_fill_scatter_body,
        out_type=jax.ShapeDtypeStruct((_B * _S, _D), jnp.float32),
        mesh=mesh,
        scratch_types=[
            pltpu.VMEM((64, _D), jnp.float32),
            pltpu.VMEM((_K,), jnp.int32),
            pltpu.VMEM((_K,), jnp.int32),
            pltpu.VMEM((_K, _D), jnp.float32),
            pltpu.SemaphoreType.DMA,
            pltpu.SemaphoreType.DMA,
        ],
    )
    return gather, fill_scatter


def _tc_body(xg_ref, wq_ref, wk_ref, wv_ref, wo_ref, bq_ref, bk_ref, bv_ref,
             bo_ref, w1_ref, b1_ref, w2_ref, b2_ref, ew_ref,
             delta_ref, fill_ref):
    dn = (((1,), (1,)), ((), ()))
    hi = lax.Precision.HIGHEST
    xs = xg_ref[0, :_EP, :]
    xd = xg_ref[0, _EP:, :]
    qs = lax.dot_general(xs, wq_ref[...], dn, precision=hi,
                         preferred_element_type=jnp.float32) + bq_ref[...]
    kd = lax.dot_general(xd, wk_ref[...], dn, precision=hi,
                         preferred_element_type=jnp.float32) + bk_ref[...]
    vd = lax.dot_general(xd, wv_ref[...], dn, precision=hi,
                         preferred_element_type=jnp.float32) + bv_ref[...]
    w1 = w1_ref[...]
    w1a = w1[:, :_DH]
    w1b = w1[:, _DH:]
    b1 = b1_ref[...]
    w2 = w2_ref[...]
    b2 = b2_ref[0, 0]
    es_cols = []
    for h in range(_H):
        sl = slice(h * _DH, (h + 1) * _DH)
        pre = (lax.dot_general(qs[:, sl], w1a, dn, precision=hi,
                               preferred_element_type=jnp.float32)
               + lax.dot_general(kd[:, sl], w1b, dn, precision=hi,
                                 preferred_element_type=jnp.float32) + b1)
        hm = 0.5 * pre * (1.0 + lax.erf(pre * (2.0 ** -0.5)))
        es_cols.append(jnp.sum(hm * w2, axis=1, keepdims=True) + b2)
    es = jnp.concatenate(es_cols, axis=1) * _SCALE
    valid = lax.broadcasted_iota(jnp.int32, (_EP, 1), 0) < _E
    es = jnp.where(valid, es, -1e30)
    m = jnp.max(es, axis=0, keepdims=True)
    p = jnp.exp(es - m)
    p = jnp.where(valid, p, 0.0)
    ea = (p / jnp.sum(p, axis=0, keepdims=True)) * ew_ref[...]
    row = jnp.concatenate(
        [ea[:, h:h + 1] * vd[:, h * _DH:(h + 1) * _DH] for h in range(_H)],
        axis=1)
    delta_ref[0] = lax.dot_general(row, wo_ref[...], dn, precision=hi,
                                   preferred_element_type=jnp.float32) \
        + bo_ref[...]
    fill_ref[...] = jnp.broadcast_to(bo_ref[...], (64, _D))


_full = lambda shape: pl.BlockSpec(shape, lambda b: (0,) * len(shape))

_tc_compute = pl.pallas_call(
    _tc_body,
    grid=(_B,),
    in_specs=[
        pl.BlockSpec((1, 2 * _EP, _D), lambda b: (b, 0, 0)),
        _full((_D, _D)), _full((_D, _D)), _full((_D, _D)), _full((_D, _D)),
        _full((1, _D)), _full((1, _D)), _full((1, _D)), _full((1, _D)),
        _full((_DH, 2 * _DH)), _full((1, _DH)), _full((1, _DH)),
        _full((1, 1)), _full((1, _H)),
    ],
    out_specs=[
        pl.BlockSpec((1, _EP, _D), lambda b: (b, 0, 0)),
        pl.BlockSpec((64, _D), lambda b: (0, 0)),
    ],
    out_shape=[
        jax.ShapeDtypeStruct((_B, _EP, _D), jnp.float32),
        jax.ShapeDtypeStruct((64, _D), jnp.float32),
    ],
)


def kernel(x, Wq, bq, Wk, bk, Wv, bv, Wo, bo, edge_weight, W1, b1, W2, b2):
    B, S, D = x.shape
    assert (B, S, D) == (_B, _S, _D)
    _sc_gather, _sc_fill_scatter = _sc_kernels()
    xg = _sc_gather(x.reshape(B * S, D), jnp.asarray(_GIDX_NP))
    delta, fill64 = _tc_compute(
        xg.reshape(B, 2 * _EP, D), Wq, Wk, Wv, Wo,
        bq.reshape(1, D), bk.reshape(1, D), bv.reshape(1, D),
        bo.reshape(1, D), W1, b1.reshape(1, _DH), W2.reshape(1, _DH),
        b2.reshape(1, 1), edge_weight.reshape(1, _H))
    out = _sc_fill_scatter(fill64, delta.reshape(B * _EP, D),
                           jnp.asarray(_DIDX_NP), jnp.asarray(_SIDX_NP))
    return out.reshape(B, S, D)


# speedup vs baseline: 7.4233x; 1.5071x over previous
import functools

import numpy as np
import jax
import jax.numpy as jnp
from jax import lax
from jax.experimental import pallas as pl
from jax.experimental.pallas import tpu as pltpu
from jax.experimental.pallas import tpu_sc as plsc

_B, _S, _D, _H, _DH = 2, 8192, 768, 12, 64
_E = 500
_EP = 512
_NW = 32
_GPW = (_B * 2 * _EP) // _NW
_RPW = (_B * _S) // _NW
_K = 32
_SCALE = _DH ** -0.5


def _halton(b, n):
    h, d = 0, 1
    seq = []
    for _ in range(n):
        x = d - h
        if x == 1:
            h = 1
            d *= b
        else:
            y = d // b
            while x <= y:
                y //= b
            h = (b + 1) * y - x
        seq.append(h / d)
    return np.array(seq, dtype=np.float64)


def _build_constants():
    n = min(500, int(0.01 * _S * _S))
    h2 = _halton(2, n)
    h3 = _halton(3, n)
    src = (h2 * _S).astype(np.int64)
    dst = (h3 * _S).astype(np.int64)
    keep = src != dst
    src = src[keep][:n]
    dst = dst[keep][:n]
    assert src.shape[0] == _E
    assert np.unique(src).size == _E

    gidx = np.zeros((_B, 2, _EP), dtype=np.int32)
    for b in range(_B):
        gidx[b, 0, :_E] = b * _S + src
        gidx[b, 0, _E:] = b * _S
        gidx[b, 1, :_E] = b * _S + dst
        gidx[b, 1, _E:] = b * _S
    gidx = gidx.reshape(-1)

    didx = np.zeros((_NW, _K), dtype=np.int32)
    sidx = np.zeros((_NW, _K), dtype=np.int32)
    for t in range(_NW):
        lo = t * _RPW
        rows = []
        for b in range(_B):
            flat = b * _S + src
            for e in np.nonzero((flat >= lo) & (flat < lo + _RPW))[0]:
                rows.append((b * _EP + int(e), int(flat[e])))
        assert 0 < len(rows) <= _K
        while len(rows) < _K:
            rows.append(rows[-1])
        didx[t] = [r[0] for r in rows]
        sidx[t] = [r[1] for r in rows]
    return gidx, didx.reshape(-1), sidx.reshape(-1)


_GIDX_NP, _DIDX_NP, _SIDX_NP = _build_constants()

def _sc_gather_body(x_hbm, idx_hbm, out_hbm, idx_v, rows_v, sem):
    wid = lax.axis_index("s") * 2 + lax.axis_index("c")
    base = wid * _GPW
    pltpu.sync_copy(idx_hbm.at[pl.ds(base, _GPW)], idx_v)
    pltpu.async_copy(x_hbm.at[idx_v], rows_v, sem).wait()
    pltpu.sync_copy(rows_v, out_hbm.at[pl.ds(base, _GPW)])


def _sc_fill_scatter_body(fill_hbm, delta_hbm, didx_hbm, sidx_hbm, out_hbm,
                          fill_v, didx_v, sidx_v, drows_v, fsem, ssem):
    wid = lax.axis_index("s") * 2 + lax.axis_index("c")
    base = wid * _RPW
    pltpu.sync_copy(fill_hbm, fill_v)
    fills = [
        pltpu.async_copy(fill_v, out_hbm.at[pl.ds(base + j * 64, 64)], fsem)
        for j in range(_RPW // 64)
    ]
    pltpu.sync_copy(didx_hbm.at[pl.ds(wid * _K, _K)], didx_v)
    pltpu.sync_copy(sidx_hbm.at[pl.ds(wid * _K, _K)], sidx_v)
    pltpu.async_copy(delta_hbm.at[didx_v], drows_v, ssem).wait()
    for c in fills:
        c.wait()
    pltpu.async_copy(drows_v, out_hbm.at[sidx_v], ssem).wait()


@functools.lru_cache(maxsize=None)
def _sc_kernels():
    mesh = plsc.VectorSubcoreMesh(core_axis_name="c", subcore_axis_name="s")
    gather = pl.kernel(
        _sc_gather_body,
        out_type=jax.ShapeDtypeStruct((_B * 2 * _EP, _D), jnp.float32),
        mesh=mesh,
        scratch_types=[
            pltpu.VMEM((_GPW,), jnp.int32),
            pltpu.VMEM((_GPW, _D), jnp.float32),
            pltpu.SemaphoreType.DMA,
        ],
    )
    fill_scatter = pl.kernel(
        _sc_fill_scatter_body,
        out_type=jax.ShapeDtypeStruct((_B * _S, _D), jnp.float32),
        mesh=mesh,
        scratch_types=[
            pltpu.VMEM((64, _D), jnp.float32),
            pltpu.VMEM((_K,), jnp.int32),
            pltpu.VMEM((_K,), jnp.int32),
            pltpu.VMEM((_K, _D), jnp.float32),
            pltpu.SemaphoreType.DMA,
            pltpu.SemaphoreType.DMA,
        ],
    )
    return gather, fill_scatter


def _tc_body(xg_ref, wq_ref, wk_ref, wv_ref, wo_ref, bq_ref, bk_ref, bv_ref,
             bo_ref, w1_ref, b1_ref, w2_ref, b2_ref, ew_ref,
             delta_ref, fill_ref):
    dn = (((1,), (1,)), ((), ()))
    hi = lax.Precision.DEFAULT
    xs = xg_ref[0, :_EP, :]
    xd = xg_ref[0, _EP:, :]
    qs = lax.dot_general(xs, wq_ref[...], dn, precision=hi,
                         preferred_element_type=jnp.float32) + bq_ref[...]
    kd = lax.dot_general(xd, wk_ref[...], dn, precision=hi,
                         preferred_element_type=jnp.float32) + bk_ref[...]
    vd = lax.dot_general(xd, wv_ref[...], dn, precision=hi,
                         preferred_element_type=jnp.float32) + bv_ref[...]
    w1 = w1_ref[...]
    w1a = w1[:, :_DH]
    w1b = w1[:, _DH:]
    b1 = b1_ref[...]
    w2 = w2_ref[...]
    b2 = b2_ref[0, 0]
    es_cols = []
    for h in range(_H):
        sl = slice(h * _DH, (h + 1) * _DH)
        pre = (lax.dot_general(qs[:, sl], w1a, dn, precision=hi,
                               preferred_element_type=jnp.float32)
               + lax.dot_general(kd[:, sl], w1b, dn, precision=hi,
                                 preferred_element_type=jnp.float32) + b1)
        hm = 0.5 * pre * (1.0 + lax.erf(pre * (2.0 ** -0.5)))
        es_cols.append(jnp.sum(hm * w2, axis=1, keepdims=True) + b2)
    es = jnp.concatenate(es_cols, axis=1) * _SCALE
    valid = lax.broadcasted_iota(jnp.int32, (_EP, 1), 0) < _E
    es = jnp.where(valid, es, -1e30)
    m = jnp.max(es, axis=0, keepdims=True)
    p = jnp.exp(es - m)
    p = jnp.where(valid, p, 0.0)
    ea = (p / jnp.sum(p, axis=0, keepdims=True)) * ew_ref[...]
    row = jnp.concatenate(
        [ea[:, h:h + 1] * vd[:, h * _DH:(h + 1) * _DH] for h in range(_H)],
        axis=1)
    delta_ref[0] = lax.dot_general(row, wo_ref[...], dn, precision=hi,
                                   preferred_element_type=jnp.float32) \
        + bo_ref[...]
    fill_ref[...] = jnp.broadcast_to(bo_ref[...], (64, _D))


_full = lambda shape: pl.BlockSpec(shape, lambda b: (0,) * len(shape))

_tc_compute = pl.pallas_call(
    _tc_body,
    grid=(_B,),
    in_specs=[
        pl.BlockSpec((1, 2 * _EP, _D), lambda b: (b, 0, 0)),
        _full((_D, _D)), _full((_D, _D)), _full((_D, _D)), _full((_D, _D)),
        _full((1, _D)), _full((1, _D)), _full((1, _D)), _full((1, _D)),
        _full((_DH, 2 * _DH)), _full((1, _DH)), _full((1, _DH)),
        _full((1, 1)), _full((1, _H)),
    ],
    out_specs=[
        pl.BlockSpec((1, _EP, _D), lambda b: (b, 0, 0)),
        pl.BlockSpec((64, _D), lambda b: (0, 0)),
    ],
    out_shape=[
        jax.ShapeDtypeStruct((_B, _EP, _D), jnp.float32),
        jax.ShapeDtypeStruct((64, _D), jnp.float32),
    ],
)


def kernel(x, Wq, bq, Wk, bk, Wv, bv, Wo, bo, edge_weight, W1, b1, W2, b2):
    B, S, D = x.shape
    assert (B, S, D) == (_B, _S, _D)
    _sc_gather, _sc_fill_scatter = _sc_kernels()
    xg = _sc_gather(x.reshape(B * S, D), jnp.asarray(_GIDX_NP))
    delta, fill64 = _tc_compute(
        xg.reshape(B, 2 * _EP, D), Wq, Wk, Wv, Wo,
        bq.reshape(1, D), bk.reshape(1, D), bv.reshape(1, D),
        bo.reshape(1, D), W1, b1.reshape(1, _DH), W2.reshape(1, _DH),
        b2.reshape(1, 1), edge_weight.reshape(1, _H))
    out = _sc_fill_scatter(fill64, delta.reshape(B * _EP, D),
                           jnp.asarray(_DIDX_NP), jnp.asarray(_SIDX_NP))
    return out.reshape(B, S, D)
